# baseline (speedup 1.0000x reference)
import functools

import jax
import jax.numpy as jnp
import numpy as np
from jax import lax
from jax.experimental import pallas as pl
from jax.experimental.pallas import tpu as pltpu
from jax.experimental.pallas import tpu_sc as plsc

_N = 100000
_D = 128
_NC = 2
_NS = 16
_NW = _NC * _NS
_C = 128
_SB = 2
_NSUP = 12
_FULL = _SB * _C * _NSUP
_PL = 56
_PS = 48
_NLONG = 20

_PERM_CACHE: dict = {}


def _row_start(w: int) -> int:
    return _FULL * w + _PL * min(w, _NLONG) + _PS * max(0, w - _NLONG)


def _perm_chunked():
    if "p" not in _PERM_CACHE:
        with jax.ensure_compile_time_eval():
            p = jax.random.permutation(jax.random.key(42), _N)
        p = np.asarray(p, dtype=np.int32)
        sup = np.zeros((_NW, _NSUP, _SB * _C), dtype=np.int32)
        part = np.zeros((_NW, 64), dtype=np.int32)
        for w in range(_NW):
            r = _row_start(w)
            sup[w] = p[r: r + _FULL].reshape(_NSUP, _SB * _C)
            n = _PL if w < _NLONG else _PS
            part[w, :n] = p[r + _FULL: r + _FULL + n]
        _PERM_CACHE["p"] = (sup, part)
    return _PERM_CACHE["p"]


@functools.partial(
    pl.kernel,
    out_type=jax.ShapeDtypeStruct((_N, _D), jnp.float32),
    mesh=plsc.VectorSubcoreMesh(core_axis_name="c", subcore_axis_name="s"),
    scratch_types=[
        pltpu.VMEM((_NSUP, _SB * _C), jnp.int32),
        pltpu.VMEM((1, 64), jnp.int32),
        pltpu.VMEM((3, _SB * _C, _D), jnp.float32),
        pltpu.SemaphoreType.DMA,
        pltpu.SemaphoreType.DMA,
    ],
)
def _permute_rows(table_hbm, idxs_hbm, idxp_hbm, out_hbm,
                  idxs_v, idxp_v, bufs, gsem, ssem):
    wid = lax.axis_index("s") * _NC + lax.axis_index("c")
    base = 8 * jnp.where(wid < _NLONG,
                         wid * (_FULL + _PL) // 8,
                         (wid * (_FULL + _PS) + _NLONG * (_PL - _PS)) // 8)
    pltpu.sync_copy(idxs_hbm.at[wid], idxs_v)
    pltpu.sync_copy(idxp_hbm.at[pl.ds(wid, 1)], idxp_v)

    _NBUF = 3
    _G = 2

    def gather_chunk(s, i):
        return pltpu.async_copy(
            table_hbm.at[idxs_v.at[s, pl.ds(i * _C, _C)]],
            bufs.at[s % _NBUF, pl.ds(i * _C, _C)], gsem)

    def store_super(s):
        return pltpu.async_copy(
            bufs.at[s % _NBUF],
            out_hbm.at[pl.ds(base + s * _SB * _C, _SB * _C)], ssem)

    gathers = {}
    stores = {}
    for s in range(_G):
        gathers[s] = [gather_chunk(s, i) for i in range(_SB)]
    for s in range(_NSUP):
        nxt = s + _G
        if nxt < _NSUP:
            prev = nxt - _NBUF
            if prev in stores:
                stores.pop(prev).wait()
            gathers[nxt] = [gather_chunk(nxt, i) for i in range(_SB)]
        for g in gathers.pop(s):
            g.wait()
        stores[s] = store_super(s)
    for s in sorted(stores):
        stores.pop(s).wait()

    def do_partial(n):
        pltpu.async_copy(
            table_hbm.at[idxp_v.at[0, pl.ds(0, n)]],
            bufs.at[0, pl.ds(0, n)], gsem).wait()
        pltpu.async_copy(
            bufs.at[0, pl.ds(0, n)],
            out_hbm.at[pl.ds(base + _FULL, n)], ssem).wait()

    @pl.when(wid < _NLONG)
    def _():
        do_partial(_PL)

    @pl.when(wid >= _NLONG)
    def _():
        do_partial(_PS)


def kernel(data):
    x = data.reshape(_N, _D)
    sup, part = _perm_chunked()
    out = _permute_rows(x, jnp.asarray(sup), jnp.asarray(part))
    return out.reshape(_N * _D)


# speedup vs baseline: 1.0384x; 1.0384x over previous
import functools

import jax
import jax.numpy as jnp
import numpy as np
from jax import lax
from jax.experimental import pallas as pl
from jax.experimental.pallas import tpu as pltpu
from jax.experimental.pallas import tpu_sc as plsc

_N = 100000
_D = 128
_NC = 2
_NS = 16
_NW = _NC * _NS
_C = 128
_SB = 3
_NSUP = 8
_FULL = _SB * _C * _NSUP
_PL = 56
_PS = 48
_NLONG = 20

_PERM_CACHE: dict = {}


def _row_start(w: int) -> int:
    return _FULL * w + _PL * min(w, _NLONG) + _PS * max(0, w - _NLONG)


def _perm_chunked():
    if "p" not in _PERM_CACHE:
        with jax.ensure_compile_time_eval():
            p = jax.random.permutation(jax.random.key(42), _N)
        p = np.asarray(p, dtype=np.int32)
        sup = np.zeros((_NW, _NSUP, _SB * _C), dtype=np.int32)
        part = np.zeros((_NW, 64), dtype=np.int32)
        for w in range(_NW):
            r = _row_start(w)
            sup[w] = p[r: r + _FULL].reshape(_NSUP, _SB * _C)
            n = _PL if w < _NLONG else _PS
            part[w, :n] = p[r + _FULL: r + _FULL + n]
        _PERM_CACHE["p"] = (sup, part)
    return _PERM_CACHE["p"]


@functools.partial(
    pl.kernel,
    out_type=jax.ShapeDtypeStruct((_N, _D), jnp.float32),
    mesh=plsc.VectorSubcoreMesh(core_axis_name="c", subcore_axis_name="s"),
    scratch_types=[
        pltpu.VMEM((_NSUP, _SB * _C), jnp.int32),
        pltpu.VMEM((1, 64), jnp.int32),
        pltpu.VMEM((2, _SB * _C, _D), jnp.float32),
        pltpu.SemaphoreType.DMA,
        pltpu.SemaphoreType.DMA,
        pltpu.SemaphoreType.DMA,
    ],
)
def _permute_rows(table_hbm, idxs_hbm, idxp_hbm, out_hbm,
                  idxs_v, idxp_v, bufs, gsem, ssem, psem):
    wid = lax.axis_index("s") * _NC + lax.axis_index("c")
    base = 8 * jnp.where(wid < _NLONG,
                         wid * (_FULL + _PL) // 8,
                         (wid * (_FULL + _PS) + _NLONG * (_PL - _PS)) // 8)
    pidx_copy = pltpu.async_copy(idxp_hbm.at[pl.ds(wid, 1)], idxp_v, psem)
    pltpu.sync_copy(idxs_hbm.at[wid], idxs_v)

    def gather_chunk(s, i):
        return pltpu.async_copy(
            table_hbm.at[idxs_v.at[s, pl.ds(i * _C, _C)]],
            bufs.at[s % 2, pl.ds(i * _C, _C)], gsem)

    def store_super(s):
        return pltpu.async_copy(
            bufs.at[s % 2],
            out_hbm.at[pl.ds(base + s * _SB * _C, _SB * _C)], ssem)

    gathers = {0: [gather_chunk(0, i) for i in range(_SB)]}
    stores = {}
    for s in range(_NSUP):
        if s + 1 < _NSUP:
            if s - 1 in stores:
                stores.pop(s - 1).wait()
            gathers[s + 1] = [gather_chunk(s + 1, i) for i in range(_SB)]
        for g in gathers.pop(s):
            g.wait()
        stores[s] = store_super(s)
    stores.pop(_NSUP - 2).wait()
    pidx_copy.wait()

    def do_partial(n):
        g = pltpu.async_copy(
            table_hbm.at[idxp_v.at[0, pl.ds(0, n)]],
            bufs.at[0, pl.ds(0, n)], gsem)
        stores[_NSUP - 1].wait()
        g.wait()
        pltpu.async_copy(
            bufs.at[0, pl.ds(0, n)],
            out_hbm.at[pl.ds(base + _FULL, n)], ssem).wait()

    @pl.when(wid < _NLONG)
    def _():
        do_partial(_PL)

    @pl.when(wid >= _NLONG)
    def _():
        do_partial(_PS)


def kernel(data):
    x = data.reshape(_N, _D)
    sup, part = _perm_chunked()
    out = _permute_rows(x, jnp.asarray(sup), jnp.asarray(part))
    return out.reshape(_N * _D)


# speedup vs baseline: 1.0406x; 1.0021x over previous
import functools

import jax
import jax.numpy as jnp
import numpy as np
from jax import lax
from jax.experimental import pallas as pl
from jax.experimental.pallas import tpu as pltpu
from jax.experimental.pallas import tpu_sc as plsc

_N = 100000
_D = 128
_NC = 2
_NS = 16
_NW = _NC * _NS
_C = 128
_SB = 3
_NSUP = 8
_FULL = _SB * _C * _NSUP
_PL = 56
_PS = 48
_NLONG = 20

_PERM_CACHE: dict = {}


def _row_start(w: int) -> int:
    return _FULL * w + _PL * min(w, _NLONG) + _PS * max(0, w - _NLONG)


def _perm_chunked():
    if "p" not in _PERM_CACHE:
        with jax.ensure_compile_time_eval():
            p = jax.random.permutation(jax.random.key(42), _N)
        p = np.asarray(p, dtype=np.int32)
        sup = np.zeros((_NW, _NSUP, _SB * _C), dtype=np.int32)
        part = np.zeros((_NW, 64), dtype=np.int32)
        for w in range(_NW):
            r = _row_start(w)
            sup[w] = p[r: r + _FULL].reshape(_NSUP, _SB * _C)
            n = _PL if w < _NLONG else _PS
            part[w, :n] = p[r + _FULL: r + _FULL + n]
        _PERM_CACHE["p"] = (sup, part)
    return _PERM_CACHE["p"]


@functools.partial(
    pl.kernel,
    out_type=jax.ShapeDtypeStruct((_N, _D), jnp.float32),
    mesh=plsc.VectorSubcoreMesh(core_axis_name="c", subcore_axis_name="s"),
    scratch_types=[
        pltpu.VMEM((_NSUP, _SB * _C), jnp.int32),
        pltpu.VMEM((1, 64), jnp.int32),
        pltpu.VMEM((2, _SB * _C, _D), jnp.float32),
        pltpu.SemaphoreType.DMA,
        pltpu.SemaphoreType.DMA,
        pltpu.SemaphoreType.DMA,
    ],
)
def _permute_rows(table_hbm, idxs_hbm, idxp_hbm, out_hbm,
                  idxs_v, idxp_v, bufs, gsem, ssem, psem):
    wid = lax.axis_index("s") * _NC + lax.axis_index("c")
    base = 8 * jnp.where(wid < _NLONG,
                         wid * (_FULL + _PL) // 8,
                         (wid * (_FULL + _PS) + _NLONG * (_PL - _PS)) // 8)
    pidx_copy = pltpu.async_copy(idxp_hbm.at[pl.ds(wid, 1)], idxp_v, psem)
    pltpu.sync_copy(idxs_hbm.at[wid, pl.ds(0, 1)], idxs_v.at[pl.ds(0, 1)])
    ridx_copy = pltpu.async_copy(
        idxs_hbm.at[wid, pl.ds(1, _NSUP - 1)],
        idxs_v.at[pl.ds(1, _NSUP - 1)], psem)

    def gather_chunk(s, i):
        return pltpu.async_copy(
            table_hbm.at[idxs_v.at[s, pl.ds(i * _C, _C)]],
            bufs.at[s % 2, pl.ds(i * _C, _C)], gsem)

    def store_super(s):
        return pltpu.async_copy(
            bufs.at[s % 2],
            out_hbm.at[pl.ds(base + s * _SB * _C, _SB * _C)], ssem)

    gathers = {0: [gather_chunk(0, i) for i in range(_SB)]}
    ridx_copy.wait()
    stores = {}
    for s in range(_NSUP):
        if s + 1 < _NSUP:
            if s - 1 in stores:
                stores.pop(s - 1).wait()
            gathers[s + 1] = [gather_chunk(s + 1, i) for i in range(_SB)]
        for g in gathers.pop(s):
            g.wait()
        stores[s] = store_super(s)
    stores.pop(_NSUP - 2).wait()
    pidx_copy.wait()

    def do_partial(n):
        g = pltpu.async_copy(
            table_hbm.at[idxp_v.at[0, pl.ds(0, n)]],
            bufs.at[0, pl.ds(0, n)], gsem)
        stores[_NSUP - 1].wait()
        g.wait()
        pltpu.async_copy(
            bufs.at[0, pl.ds(0, n)],
            out_hbm.at[pl.ds(base + _FULL, n)], ssem).wait()

    @pl.when(wid < _NLONG)
    def _():
        do_partial(_PL)

    @pl.when(wid >= _NLONG)
    def _():
        do_partial(_PS)


def kernel(data):
    x = data.reshape(_N, _D)
    sup, part = _perm_chunked()
    out = _permute_rows(x, jnp.asarray(sup), jnp.asarray(part))
    return out.reshape(_N * _D)
